# Initial kernel scaffold; baseline (speedup 1.0000x reference)
import jax
import jax.numpy as jnp
from jax.experimental import pallas as pl


def kernel(x, edge_index, edge_attr, edge_t, edge_d, node_W1, node_b1, node_W2, node_b2, edge_table, dire_table, l0_qW, l0_qb, l0_kW, l0_kb, l0_vW, l0_vb, l0_gamma, l0_beta, l0_f1W, l0_f1b, l0_f2W, l0_f2b, l1_qW, l1_qb, l1_kW, l1_kb, l1_vW, l1_vb, l1_gamma, l1_beta, l1_f1W, l1_f1b, l1_f2W, l1_f2b, agg_W, agg_b, pred_W, pred_b):
    raise NotImplementedError("write your pallas kernel here")


# speedup vs baseline: 34.3122x; 34.3122x over previous
import functools

import jax
import jax.numpy as jnp
from jax import lax
from jax.experimental import pallas as pl
from jax.experimental.pallas import tpu as pltpu
from jax.experimental.pallas import tpu_sc as plsc

N = 10000
E = 320000
IN = 128
HID = 128
HEADS = 16
HD = HID // HEADS
SCALE = HD ** -0.5
EPS = 1e-5
EC = 16
OUT = 64

NC = 2
NS = 16
NW = NC * NS
EPW = E // NW
CH = 80
NCHUNK = EPW // CH
WRCH = 200
NWR = N // WRCH
WR_PER_TILE = (NWR + NS - 1) // NS
NZ = N // CH
Z_PER_TILE = (NZ + NS - 1) // NS

BN_BLK = 2000
BE_BLK = 8000


def _gelu(x):
    return 0.5 * x * (1.0 + lax.erf(x * (2.0 ** -0.5)))


def _embed_body(x_ref, w1, b1, w2, b2, qw, qb, kw, kb, vw, vb,
                h_ref, q_ref, k_ref, v_ref):
    f32 = jnp.float32
    h = _gelu(jnp.dot(x_ref[...], w1[...], preferred_element_type=f32) + b1[...])
    h = _gelu(jnp.dot(h, w2[...], preferred_element_type=f32) + b2[...])
    h_ref[...] = h
    q_ref[...] = (jnp.dot(h, qw[...], preferred_element_type=f32) + qb[...]) * SCALE
    k_ref[...] = jnp.dot(h, kw[...], preferred_element_type=f32) + kb[...]
    v_ref[...] = jnp.dot(h, vw[...], preferred_element_type=f32) + vb[...]


def _embed_call(x, w1, b1, w2, b2, qw, qb, kw, kb, vw, vb):
    nb = N // BN_BLK
    blk = lambda r, c: pl.BlockSpec((r, c), lambda i: (0, 0))
    row_blk = pl.BlockSpec((BN_BLK, HID), lambda i: (i, 0))
    return pl.pallas_call(
        _embed_body,
        grid=(nb,),
        in_specs=[row_blk,
                  blk(IN, HID), blk(1, HID), blk(HID, HID), blk(1, HID),
                  blk(HID, HID), blk(1, HID), blk(HID, HID), blk(1, HID),
                  blk(HID, HID), blk(1, HID)],
        out_specs=[row_blk, row_blk, row_blk, row_blk],
        out_shape=[jax.ShapeDtypeStruct((N, HID), jnp.float32)] * 4,
    )(x, w1, b1, w2, b2, qw, qb, kw, kb, vw, vb)


def _ew_body(attr_ref, d_ref, t_ref, et_ref, dt_ref, tw_ref, ew_ref):
    oh = (attr_ref[...] == lax.broadcasted_iota(jnp.int32, (BE_BLK, EC), 1))
    ew1 = jnp.dot(oh.astype(jnp.float32), et_ref[...],
                  preferred_element_type=jnp.float32)
    dt = dt_ref[...]
    dire = dt[0:1, :] + d_ref[...].astype(jnp.float32) * (dt[1:2, :] - dt[0:1, :])
    te = jnp.cos(t_ref[...] * tw_ref[...])
    ew_ref[...] = (ew1 + dire) * te


def _ew_call(edge_attr, edge_d, edge_t, edge_table, dire_table, tw):
    nb = E // BE_BLK
    col_blk = pl.BlockSpec((BE_BLK, 1), lambda i: (i, 0))
    blk = lambda r, c: pl.BlockSpec((r, c), lambda i: (0, 0))
    return pl.pallas_call(
        _ew_body,
        grid=(nb,),
        in_specs=[col_blk, col_blk, col_blk,
                  blk(EC, HEADS), blk(2, HEADS), blk(1, HEADS)],
        out_specs=pl.BlockSpec((BE_BLK, HEADS), lambda i: (i, 0)),
        out_shape=jax.ShapeDtypeStruct((E, HEADS), jnp.float32),
    )(edge_attr.reshape(E, 1), edge_d.reshape(E, 1), edge_t.reshape(E, 1),
      edge_table, dire_table, tw)


def _sc_edge_body(q_hbm, k_hbm, v_hbm, row_hbm, col_hbm, ew_hbm, out_hbm,
                  row_v, col_v, qr, kr, vr, ewc, agg_sh,
                  sem_q, sem_k, sem_v):
    cid = lax.axis_index("c")
    sid = lax.axis_index("s")
    wid = cid * NS + sid

    def _zb(i, _):
        for dd in range(HID // 16):
            vr[i, pl.ds(dd * 16, 16)] = jnp.zeros((16,), jnp.float32)
        return 0
    lax.fori_loop(0, CH, _zb, 0)

    def _zero_chunk(j):
        @pl.when(j < NZ)
        def _():
            pltpu.sync_copy(vr, agg_sh.at[pl.ds(j * CH, CH)])
    for jj in range(Z_PER_TILE):
        _zero_chunk(sid + jj * NS)
    plsc.subcore_barrier()

    base_w = wid * EPW

    def _chunk(ci, _):
        base = base_w + ci * CH
        pltpu.sync_copy(row_hbm.at[pl.ds(base, CH)], row_v)
        pltpu.sync_copy(col_hbm.at[pl.ds(base, CH)], col_v)
        pltpu.sync_copy(ew_hbm.at[pl.ds(base, CH)], ewc)
        cq = pltpu.async_copy(q_hbm.at[row_v], qr, sem_q)
        ck = pltpu.async_copy(k_hbm.at[col_v], kr, sem_k)
        cv = pltpu.async_copy(v_hbm.at[col_v], vr, sem_v)
        cq.wait()
        ck.wait()
        cv.wait()

        def _edge(e, _):
            acc = qr[e, pl.ds(0, 16)] * kr[e, pl.ds(0, 16)]
            for dd in range(1, HID // 16):
                acc = acc + qr[e, pl.ds(dd * 16, 16)] * kr[e, pl.ds(dd * 16, 16)]
            a = ewc[e, :] / (1.0 + jnp.exp(-acc))
            for dd in range(HID // 16):
                vr[e, pl.ds(dd * 16, 16)] = a * vr[e, pl.ds(dd * 16, 16)]
            return 0
        lax.fori_loop(0, CH, _edge, 0)
        pltpu.sync_copy(vr, agg_sh.at[row_v], add=True)
        return 0
    lax.fori_loop(0, NCHUNK, _chunk, 0)

    plsc.subcore_barrier()

    def _write_chunk(j):
        @pl.when(j < NWR)
        def _():
            pltpu.sync_copy(agg_sh.at[pl.ds(j * WRCH, WRCH)],
                            out_hbm.at[cid, pl.ds(j * WRCH, WRCH)])
    for jj in range(WR_PER_TILE):
        _write_chunk(sid + jj * NS)


@functools.lru_cache(maxsize=1)
def _sc_edge_build():
    return pl.kernel(
        _sc_edge_body,
        out_type=jax.ShapeDtypeStruct((NC, N, HID), jnp.float32),
        mesh=plsc.VectorSubcoreMesh(core_axis_name="c", subcore_axis_name="s",
                                    num_cores=NC, num_subcores=NS),
        scratch_types=[
            pltpu.VMEM((CH,), jnp.int32),
            pltpu.VMEM((CH,), jnp.int32),
            pltpu.VMEM((CH, HID), jnp.float32),
            pltpu.VMEM((CH, HID), jnp.float32),
            pltpu.VMEM((CH, HID), jnp.float32),
            pltpu.VMEM((CH, HEADS), jnp.float32),
            pltpu.VMEM_SHARED((N, HID), jnp.float32),
            pltpu.SemaphoreType.DMA,
            pltpu.SemaphoreType.DMA,
            pltpu.SemaphoreType.DMA,
        ],
    )


def _sc_edge(q, k, v, row, col, ew):
    return _sc_edge_build()(q, k, v, row, col, ew)


def _stats_body(h_ref, a0_ref, a1_ref, hn_ref, s_ref):
    hn = h_ref[...] + a0_ref[...] + a1_ref[...]
    hn_ref[...] = hn
    blk = jnp.concatenate(
        [jnp.sum(hn, axis=0, keepdims=True),
         jnp.sum(hn * hn, axis=0, keepdims=True),
         jnp.zeros((6, HID), jnp.float32)], axis=0)

    @pl.when(pl.program_id(0) == 0)
    def _():
        s_ref[...] = blk

    @pl.when(pl.program_id(0) != 0)
    def _():
        s_ref[...] = s_ref[...] + blk


def _stats_call(h, a0, a1):
    nb = N // BN_BLK
    row_blk = pl.BlockSpec((BN_BLK, HID), lambda i: (i, 0))
    return pl.pallas_call(
        _stats_body,
        grid=(nb,),
        in_specs=[row_blk, row_blk, row_blk],
        out_specs=[row_blk, pl.BlockSpec((8, HID), lambda i: (0, 0))],
        out_shape=[jax.ShapeDtypeStruct((N, HID), jnp.float32),
                   jax.ShapeDtypeStruct((8, HID), jnp.float32)],
    )(h, a0, a1)


def _norm(hn, s, g, b):
    mu = s[0:1, :] * (1.0 / N)
    var = s[1:2, :] * (1.0 / N) - mu * mu
    return (hn - mu) * (lax.rsqrt(var + EPS) * g) + b


def _ffn_mid_body(hn_ref, s_ref, g_ref, b_ref, f1w, f1b, f2w, f2b,
                  qw, qb, kw, kb, vw, vb, h_ref, q_ref, k_ref, v_ref):
    f32 = jnp.float32
    xn = _norm(hn_ref[...], s_ref[...], g_ref[...], b_ref[...])
    mid = _gelu(jnp.dot(xn, f1w[...], preferred_element_type=f32) + f1b[...])
    h = jnp.dot(mid, f2w[...], preferred_element_type=f32) + f2b[...]
    h_ref[...] = h
    q_ref[...] = (jnp.dot(h, qw[...], preferred_element_type=f32) + qb[...]) * SCALE
    k_ref[...] = jnp.dot(h, kw[...], preferred_element_type=f32) + kb[...]
    v_ref[...] = jnp.dot(h, vw[...], preferred_element_type=f32) + vb[...]


def _ffn_mid_call(hn, s, g, b, f1w, f1b, f2w, f2b, qw, qb, kw, kb, vw, vb):
    nb = N // BN_BLK
    row_blk = pl.BlockSpec((BN_BLK, HID), lambda i: (i, 0))
    blk = lambda r, c: pl.BlockSpec((r, c), lambda i: (0, 0))
    return pl.pallas_call(
        _ffn_mid_body,
        grid=(nb,),
        in_specs=[row_blk, blk(8, HID), blk(1, HID), blk(1, HID),
                  blk(HID, 2 * HID), blk(1, 2 * HID), blk(2 * HID, HID), blk(1, HID),
                  blk(HID, HID), blk(1, HID), blk(HID, HID), blk(1, HID),
                  blk(HID, HID), blk(1, HID)],
        out_specs=[row_blk, row_blk, row_blk, row_blk],
        out_shape=[jax.ShapeDtypeStruct((N, HID), jnp.float32)] * 4,
    )(hn, s, g, b, f1w, f1b, f2w, f2b, qw, qb, kw, kb, vw, vb)


def _ffn_out_body(hn_ref, s_ref, g_ref, b_ref, f1w, f1b, f2w, f2b,
                  aw, ab, pw, pb, o_ref):
    f32 = jnp.float32
    xn = _norm(hn_ref[...], s_ref[...], g_ref[...], b_ref[...])
    mid = _gelu(jnp.dot(xn, f1w[...], preferred_element_type=f32) + f1b[...])
    h = jnp.dot(mid, f2w[...], preferred_element_type=f32) + f2b[...]
    xa = jnp.dot(h, aw[...], preferred_element_type=f32) + ab[...]
    o_ref[...] = jnp.dot(xa, pw[...], preferred_element_type=f32) + pb[...]


def _ffn_out_call(hn, s, g, b, f1w, f1b, f2w, f2b, aw, ab, pw, pb):
    nb = N // BN_BLK
    row_blk = pl.BlockSpec((BN_BLK, HID), lambda i: (i, 0))
    blk = lambda r, c: pl.BlockSpec((r, c), lambda i: (0, 0))
    return pl.pallas_call(
        _ffn_out_body,
        grid=(nb,),
        in_specs=[row_blk, blk(8, HID), blk(1, HID), blk(1, HID),
                  blk(HID, 2 * HID), blk(1, 2 * HID), blk(2 * HID, HID), blk(1, HID),
                  blk(HID, HID), blk(1, HID), blk(HID, OUT), blk(1, OUT)],
        out_specs=pl.BlockSpec((BN_BLK, OUT), lambda i: (i, 0)),
        out_shape=jax.ShapeDtypeStruct((N, OUT), jnp.float32),
    )(hn, s, g, b, f1w, f1b, f2w, f2b, aw, ab, pw, pb)


def kernel(x, edge_index, edge_attr, edge_t, edge_d,
           node_W1, node_b1, node_W2, node_b2, edge_table, dire_table,
           l0_qW, l0_qb, l0_kW, l0_kb, l0_vW, l0_vb, l0_gamma, l0_beta,
           l0_f1W, l0_f1b, l0_f2W, l0_f2b,
           l1_qW, l1_qb, l1_kW, l1_kb, l1_vW, l1_vb, l1_gamma, l1_beta,
           l1_f1W, l1_f1b, l1_f2W, l1_f2b,
           agg_W, agg_b, pred_W, pred_b):
    r1 = lambda a: a.reshape(1, -1)
    row = edge_index[0]
    col = edge_index[1]
    tw = (1.0 / (10.0 ** jnp.linspace(0.0, 9.0, HEADS))).astype(jnp.float32)

    ew = _ew_call(edge_attr, edge_d, edge_t, edge_table, dire_table, r1(tw))
    h0, q0, k0, v0 = _embed_call(x, node_W1, r1(node_b1), node_W2, r1(node_b2),
                                 l0_qW, r1(l0_qb), l0_kW, r1(l0_kb),
                                 l0_vW, r1(l0_vb))
    agg0 = _sc_edge(q0, k0, v0, row, col, ew)
    hn0, s0 = _stats_call(h0, agg0[0], agg0[1])
    h1, q1, k1, v1 = _ffn_mid_call(hn0, s0, r1(l0_gamma), r1(l0_beta),
                                   l0_f1W, r1(l0_f1b), l0_f2W, r1(l0_f2b),
                                   l1_qW, r1(l1_qb), l1_kW, r1(l1_kb),
                                   l1_vW, r1(l1_vb))
    agg1 = _sc_edge(q1, k1, v1, row, col, ew)
    hn1, s1 = _stats_call(h1, agg1[0], agg1[1])
    return _ffn_out_call(hn1, s1, r1(l1_gamma), r1(l1_beta),
                         l1_f1W, r1(l1_f1b), l1_f2W, r1(l1_f2b),
                         agg_W, r1(agg_b), pred_W, r1(pred_b))


# speedup vs baseline: 46.3561x; 1.3510x over previous
import functools

import jax
import jax.numpy as jnp
from jax import lax
from jax.experimental import pallas as pl
from jax.experimental.pallas import tpu as pltpu
from jax.experimental.pallas import tpu_sc as plsc

N = 10000
E = 320000
IN = 128
HID = 128
HEADS = 16
HD = HID // HEADS
SCALE = HD ** -0.5
EPS = 1e-5
EC = 16
OUT = 64

NC = 2
NS = 16
NW = NC * NS
EPW = E // NW
CH = 64
NCHUNK = EPW // CH
TAIL = EPW - NCHUNK * CH
WRCH = 200
NWR = N // WRCH
WR_PER_TILE = (NWR + NS - 1) // NS
NZ = N // CH
Z_PER_TILE = (NZ + NS - 1) // NS

BN_BLK = 2000
BE_BLK = 32000


def _gelu(x):
    return 0.5 * x * (1.0 + lax.erf(x * (2.0 ** -0.5)))


def _embed_body(x_ref, w1, b1, w2, b2, qw, qb, kw, kb, vw, vb,
                h_ref, q_ref, k_ref, v_ref):
    f32 = jnp.float32
    h = _gelu(jnp.dot(x_ref[...], w1[...], preferred_element_type=f32) + b1[...])
    h = _gelu(jnp.dot(h, w2[...], preferred_element_type=f32) + b2[...])
    h_ref[...] = h
    q_ref[...] = (jnp.dot(h, qw[...], preferred_element_type=f32) + qb[...]) * SCALE
    k_ref[...] = jnp.dot(h, kw[...], preferred_element_type=f32) + kb[...]
    v_ref[...] = jnp.dot(h, vw[...], preferred_element_type=f32) + vb[...]


def _embed_call(x, w1, b1, w2, b2, qw, qb, kw, kb, vw, vb):
    nb = N // BN_BLK
    blk = lambda r, c: pl.BlockSpec((r, c), lambda i: (0, 0))
    row_blk = pl.BlockSpec((BN_BLK, HID), lambda i: (i, 0))
    return pl.pallas_call(
        _embed_body,
        grid=(nb,),
        in_specs=[row_blk,
                  blk(IN, HID), blk(1, HID), blk(HID, HID), blk(1, HID),
                  blk(HID, HID), blk(1, HID), blk(HID, HID), blk(1, HID),
                  blk(HID, HID), blk(1, HID)],
        out_specs=[row_blk, row_blk, row_blk, row_blk],
        out_shape=[jax.ShapeDtypeStruct((N, HID), jnp.float32)] * 4,
    )(x, w1, b1, w2, b2, qw, qb, kw, kb, vw, vb)


def _ew_body(attr_ref, d_ref, t_ref, etT_ref, dtT_ref, twc_ref, ew_ref):
    attr = attr_ref[...]
    etT = etT_ref[...]
    acc = jnp.zeros((HEADS, BE_BLK), jnp.float32)
    for c in range(EC):
        acc = acc + jnp.where(attr == c, etT[:, c:c + 1], 0.0)
    dtT = dtT_ref[...]
    dire = dtT[:, 0:1] + d_ref[...].astype(jnp.float32) * (dtT[:, 1:2] - dtT[:, 0:1])
    te = jnp.cos(t_ref[...] * twc_ref[...])
    ew_ref[...] = (acc + dire) * te


def _ew_call(edge_attr, edge_d, edge_t, edge_table, dire_table, tw):
    nb = E // BE_BLK
    row_in = pl.BlockSpec((1, BE_BLK), lambda i: (0, i))
    blk = lambda r, c: pl.BlockSpec((r, c), lambda i: (0, 0))
    ewT = pl.pallas_call(
        _ew_body,
        grid=(nb,),
        in_specs=[row_in, row_in, row_in,
                  blk(HEADS, EC), blk(HEADS, 2), blk(HEADS, 1)],
        out_specs=pl.BlockSpec((HEADS, BE_BLK), lambda i: (0, i)),
        out_shape=jax.ShapeDtypeStruct((HEADS, E), jnp.float32),
    )(edge_attr.reshape(1, E), edge_d.reshape(1, E), edge_t.reshape(1, E),
      edge_table.T, dire_table.T, tw.reshape(HEADS, 1))
    return ewT.T


def _sc_edge_body(q_hbm, k_hbm, v_hbm, row_hbm, col_hbm, ew_hbm, out_hbm,
                  rowb, colb, rct, qb, kb, vb, ewc, attnb, agg_sh,
                  sq0, sq1, sk0, sk1, sem_v):
    cid = lax.axis_index("c")
    sid = lax.axis_index("s")
    wid = cid * NS + sid
    sq = (sq0, sq1)
    sk = (sk0, sk1)

    def _zrow(i, _):
        for dd in range(HID // 16):
            vb[i, pl.ds(dd * 16, 16)] = jnp.zeros((16,), jnp.float32)
        return 0
    lax.fori_loop(0, CH, _zrow, 0)

    def _zero_chunk(j):
        @pl.when(j < NZ)
        def _():
            pltpu.sync_copy(vb, agg_sh.at[pl.ds(j * CH, CH)])
    for jj in range(Z_PER_TILE):
        _zero_chunk(sid + jj * NS)

    @pl.when(sid == 0)
    def _():
        pltpu.sync_copy(vb.at[pl.ds(0, N - NZ * CH)],
                        agg_sh.at[pl.ds(NZ * CH, N - NZ * CH)])
    plsc.subcore_barrier()

    base_w = wid * EPW

    def _prefetch(gn, pn):
        @pl.when(gn < NCHUNK)
        def _():
            base = base_w + gn * CH
            pltpu.sync_copy(row_hbm.at[pl.ds(base, CH)], rowb.at[pn])
            pltpu.sync_copy(col_hbm.at[pl.ds(base, CH)], colb.at[pn])
            pltpu.async_copy(q_hbm.at[rowb.at[pn]], qb.at[pn], sq[pn])
            pltpu.async_copy(k_hbm.at[colb.at[pn]], kb.at[pn], sk[pn])

    def _attn_pass(qr, kr, ewr, nrows):
        def _edge(e, _):
            acc = qr[e, pl.ds(0, 16)] * kr[e, pl.ds(0, 16)]
            for dd in range(1, HID // 16):
                acc = acc + qr[e, pl.ds(dd * 16, 16)] * kr[e, pl.ds(dd * 16, 16)]
            attnb[pl.ds(e * HEADS, 16)] = (
                ewr[pl.ds(e * HEADS, 16)] / (1.0 + jnp.exp(-acc)))
            return 0
        lax.fori_loop(0, nrows, _edge, 0)

    def _msg_pass(vr, nrows):
        def _edge(e, _):
            a = attnb[pl.ds(e * HEADS, 16)]
            for dd in range(HID // 16):
                vr[e, pl.ds(dd * 16, 16)] = a * vr[e, pl.ds(dd * 16, 16)]
            return 0
        lax.fori_loop(0, nrows, _edge, 0)

    _prefetch(0, 0)

    def _pair(i2, _):
        for p in (0, 1):
            g = i2 * 2 + p
            _prefetch(g + 1, 1 - p)
            pltpu.make_async_copy(q_hbm.at[rowb.at[p]], qb.at[p], sq[p]).wait()
            pltpu.make_async_copy(k_hbm.at[colb.at[p]], kb.at[p], sk[p]).wait()
            cv = pltpu.async_copy(v_hbm.at[colb.at[p]], vb, sem_v)
            pltpu.sync_copy(
                ew_hbm.at[pl.ds((base_w + g * CH) * HEADS, CH * HEADS)], ewc)
            _attn_pass(qb.at[p], kb.at[p], ewc, CH)
            cv.wait()
            _msg_pass(vb, CH)
            pltpu.sync_copy(vb, agg_sh.at[rowb.at[p]], add=True)
        return 0
    lax.fori_loop(0, NCHUNK // 2, _pair, 0)

    tbase = base_w + NCHUNK * CH
    pltpu.sync_copy(row_hbm.at[pl.ds(tbase, TAIL)], rct.at[0])
    pltpu.sync_copy(col_hbm.at[pl.ds(tbase, TAIL)], rct.at[1])
    pltpu.sync_copy(ew_hbm.at[pl.ds(tbase * HEADS, TAIL * HEADS)],
                    ewc.at[pl.ds(0, TAIL * HEADS)])
    pltpu.async_copy(q_hbm.at[rct.at[0]], qb.at[0, pl.ds(0, TAIL)], sq0).wait()
    pltpu.async_copy(k_hbm.at[rct.at[1]], kb.at[0, pl.ds(0, TAIL)], sk0).wait()
    pltpu.async_copy(v_hbm.at[rct.at[1]], vb.at[pl.ds(0, TAIL)], sem_v).wait()
    _attn_pass(qb.at[0], kb.at[0], ewc, TAIL)
    _msg_pass(vb, TAIL)
    pltpu.sync_copy(vb.at[pl.ds(0, TAIL)], agg_sh.at[rct.at[0]], add=True)

    plsc.subcore_barrier()

    def _write_chunk(j):
        @pl.when(j < NWR)
        def _():
            pltpu.sync_copy(agg_sh.at[pl.ds(j * WRCH, WRCH)],
                            out_hbm.at[cid, pl.ds(j * WRCH, WRCH)])
    for jj in range(WR_PER_TILE):
        _write_chunk(sid + jj * NS)


@functools.lru_cache(maxsize=1)
def _sc_edge_build():
    return pl.kernel(
        _sc_edge_body,
        out_type=jax.ShapeDtypeStruct((NC, N, HID), jnp.float32),
        mesh=plsc.VectorSubcoreMesh(core_axis_name="c", subcore_axis_name="s",
                                    num_cores=NC, num_subcores=NS),
        scratch_types=[
            pltpu.VMEM((2, CH), jnp.int32),
            pltpu.VMEM((2, CH), jnp.int32),
            pltpu.VMEM((2, TAIL), jnp.int32),
            pltpu.VMEM((2, CH, HID), jnp.float32),
            pltpu.VMEM((2, CH, HID), jnp.float32),
            pltpu.VMEM((CH, HID), jnp.float32),
            pltpu.VMEM((CH * HEADS,), jnp.float32),
            pltpu.VMEM((CH * HEADS,), jnp.float32),
            pltpu.VMEM_SHARED((N, HID), jnp.float32),
            pltpu.SemaphoreType.DMA,
            pltpu.SemaphoreType.DMA,
            pltpu.SemaphoreType.DMA,
            pltpu.SemaphoreType.DMA,
            pltpu.SemaphoreType.DMA,
        ],
    )


def _sc_edge(q, k, v, row, col, ew):
    return _sc_edge_build()(q, k, v, row, col, ew.reshape(E * HEADS))


def _stats_body(h_ref, a0_ref, a1_ref, hn_ref, s_ref):
    hn = h_ref[...] + a0_ref[...] + a1_ref[...]
    hn_ref[...] = hn
    blk = jnp.concatenate(
        [jnp.sum(hn, axis=0, keepdims=True),
         jnp.sum(hn * hn, axis=0, keepdims=True),
         jnp.zeros((6, HID), jnp.float32)], axis=0)

    @pl.when(pl.program_id(0) == 0)
    def _():
        s_ref[...] = blk

    @pl.when(pl.program_id(0) != 0)
    def _():
        s_ref[...] = s_ref[...] + blk


def _stats_call(h, a0, a1):
    nb = N // BN_BLK
    row_blk = pl.BlockSpec((BN_BLK, HID), lambda i: (i, 0))
    return pl.pallas_call(
        _stats_body,
        grid=(nb,),
        in_specs=[row_blk, row_blk, row_blk],
        out_specs=[row_blk, pl.BlockSpec((8, HID), lambda i: (0, 0))],
        out_shape=[jax.ShapeDtypeStruct((N, HID), jnp.float32),
                   jax.ShapeDtypeStruct((8, HID), jnp.float32)],
    )(h, a0, a1)


def _norm(hn, s, g, b):
    mu = s[0:1, :] * (1.0 / N)
    var = s[1:2, :] * (1.0 / N) - mu * mu
    return (hn - mu) * (lax.rsqrt(var + EPS) * g) + b


def _ffn_mid_body(hn_ref, s_ref, g_ref, b_ref, f1w, f1b, f2w, f2b,
                  qw, qb, kw, kb, vw, vb, h_ref, q_ref, k_ref, v_ref):
    f32 = jnp.float32
    xn = _norm(hn_ref[...], s_ref[...], g_ref[...], b_ref[...])
    mid = _gelu(jnp.dot(xn, f1w[...], preferred_element_type=f32) + f1b[...])
    h = jnp.dot(mid, f2w[...], preferred_element_type=f32) + f2b[...]
    h_ref[...] = h
    q_ref[...] = (jnp.dot(h, qw[...], preferred_element_type=f32) + qb[...]) * SCALE
    k_ref[...] = jnp.dot(h, kw[...], preferred_element_type=f32) + kb[...]
    v_ref[...] = jnp.dot(h, vw[...], preferred_element_type=f32) + vb[...]


def _ffn_mid_call(hn, s, g, b, f1w, f1b, f2w, f2b, qw, qb, kw, kb, vw, vb):
    nb = N // BN_BLK
    row_blk = pl.BlockSpec((BN_BLK, HID), lambda i: (i, 0))
    blk = lambda r, c: pl.BlockSpec((r, c), lambda i: (0, 0))
    return pl.pallas_call(
        _ffn_mid_body,
        grid=(nb,),
        in_specs=[row_blk, blk(8, HID), blk(1, HID), blk(1, HID),
                  blk(HID, 2 * HID), blk(1, 2 * HID), blk(2 * HID, HID), blk(1, HID),
                  blk(HID, HID), blk(1, HID), blk(HID, HID), blk(1, HID),
                  blk(HID, HID), blk(1, HID)],
        out_specs=[row_blk, row_blk, row_blk, row_blk],
        out_shape=[jax.ShapeDtypeStruct((N, HID), jnp.float32)] * 4,
    )(hn, s, g, b, f1w, f1b, f2w, f2b, qw, qb, kw, kb, vw, vb)


def _ffn_out_body(hn_ref, s_ref, g_ref, b_ref, f1w, f1b, f2w, f2b,
                  aw, ab, pw, pb, o_ref):
    f32 = jnp.float32
    xn = _norm(hn_ref[...], s_ref[...], g_ref[...], b_ref[...])
    mid = _gelu(jnp.dot(xn, f1w[...], preferred_element_type=f32) + f1b[...])
    h = jnp.dot(mid, f2w[...], preferred_element_type=f32) + f2b[...]
    xa = jnp.dot(h, aw[...], preferred_element_type=f32) + ab[...]
    o_ref[...] = jnp.dot(xa, pw[...], preferred_element_type=f32) + pb[...]


def _ffn_out_call(hn, s, g, b, f1w, f1b, f2w, f2b, aw, ab, pw, pb):
    nb = N // BN_BLK
    row_blk = pl.BlockSpec((BN_BLK, HID), lambda i: (i, 0))
    blk = lambda r, c: pl.BlockSpec((r, c), lambda i: (0, 0))
    return pl.pallas_call(
        _ffn_out_body,
        grid=(nb,),
        in_specs=[row_blk, blk(8, HID), blk(1, HID), blk(1, HID),
                  blk(HID, 2 * HID), blk(1, 2 * HID), blk(2 * HID, HID), blk(1, HID),
                  blk(HID, HID), blk(1, HID), blk(HID, OUT), blk(1, OUT)],
        out_specs=pl.BlockSpec((BN_BLK, OUT), lambda i: (i, 0)),
        out_shape=jax.ShapeDtypeStruct((N, OUT), jnp.float32),
    )(hn, s, g, b, f1w, f1b, f2w, f2b, aw, ab, pw, pb)


def kernel(x, edge_index, edge_attr, edge_t, edge_d,
           node_W1, node_b1, node_W2, node_b2, edge_table, dire_table,
           l0_qW, l0_qb, l0_kW, l0_kb, l0_vW, l0_vb, l0_gamma, l0_beta,
           l0_f1W, l0_f1b, l0_f2W, l0_f2b,
           l1_qW, l1_qb, l1_kW, l1_kb, l1_vW, l1_vb, l1_gamma, l1_beta,
           l1_f1W, l1_f1b, l1_f2W, l1_f2b,
           agg_W, agg_b, pred_W, pred_b):
    r1 = lambda a: a.reshape(1, -1)
    row = edge_index[0]
    col = edge_index[1]
    tw = (1.0 / (10.0 ** jnp.linspace(0.0, 9.0, HEADS))).astype(jnp.float32)

    ew = _ew_call(edge_attr, edge_d, edge_t, edge_table, dire_table, r1(tw))
    h0, q0, k0, v0 = _embed_call(x, node_W1, r1(node_b1), node_W2, r1(node_b2),
                                 l0_qW, r1(l0_qb), l0_kW, r1(l0_kb),
                                 l0_vW, r1(l0_vb))
    agg0 = _sc_edge(q0, k0, v0, row, col, ew)
    hn0, s0 = _stats_call(h0, agg0[0], agg0[1])
    h1, q1, k1, v1 = _ffn_mid_call(hn0, s0, r1(l0_gamma), r1(l0_beta),
                                   l0_f1W, r1(l0_f1b), l0_f2W, r1(l0_f2b),
                                   l1_qW, r1(l1_qb), l1_kW, r1(l1_kb),
                                   l1_vW, r1(l1_vb))
    agg1 = _sc_edge(q1, k1, v1, row, col, ew)
    hn1, s1 = _stats_call(h1, agg1[0], agg1[1])
    return _ffn_out_call(hn1, s1, r1(l1_gamma), r1(l1_beta),
                         l1_f1W, r1(l1_f1b), l1_f2W, r1(l1_f2b),
                         agg_W, r1(agg_b), pred_W, r1(pred_b))
